# baseline (speedup 1.0000x reference)
import functools

import jax
import jax.numpy as jnp
from jax import lax
from jax.experimental import pallas as pl
from jax.experimental.pallas import tpu as pltpu
from jax.experimental.pallas import tpu_sc as plsc

N = 100000
D = 128
G = 4096
H_OUT = 12

NC = 2
NS = 16
NW = NC * NS
CHUNK = 128
NFULL = N // CHUNK
TAIL = N - NFULL * CHUNK
JMAX = 25
NCH = NW * JMAX
ROWS_PER_SID = G // NS
NV = D // 16
GARBAGE = G
GACC = G + CHUNK


def _sc_segment_sum(node_feats, slots_t, idx_t, zrow):
    mesh = plsc.VectorSubcoreMesh(
        core_axis_name="c", subcore_axis_name="s", num_cores=NC, num_subcores=NS
    )

    @functools.partial(
        pl.kernel,
        out_type=jax.ShapeDtypeStruct((NC, G, D), jnp.float32),
        mesh=mesh,
        scratch_types=[
            pltpu.VMEM((JMAX, CHUNK), jnp.int32),
            pltpu.VMEM((JMAX, CHUNK), jnp.int32),
            pltpu.VMEM((CHUNK + 32,), jnp.int32),
            pltpu.VMEM((3, CHUNK, D), jnp.float32),
            pltpu.VMEM((CHUNK, D), jnp.float32),
            pltpu.VMEM((CHUNK, D), jnp.float32),
            pltpu.VMEM((16,), jnp.int32),
            pltpu.VMEM((32,), jnp.int32),
            pltpu.VMEM((64,), jnp.int32),
            pltpu.VMEM((CHUNK,), jnp.int32),
            pltpu.VMEM_SHARED((GACC, D), jnp.float32),
            pltpu.SemaphoreType.DMA((3,)),
            pltpu.SemaphoreType.DMA,
        ],
    )
    def seg_sum(node_hbm, slots_hbm, idx_hbm, zrow_hbm, out_hbm,
                slots_buf, ids_buf, idsC, rbuf, zbuf, cbuf,
                ids16, ids32, ids64, ids128, acc, sem, sem_s):
        cid = lax.axis_index("c")
        sid = lax.axis_index("s")
        wid = cid * NS + sid

        pltpu.sync_copy(slots_hbm.at[wid], slots_buf)
        pltpu.sync_copy(idx_hbm.at[wid], ids_buf)
        pltpu.sync_copy(zrow_hbm, zbuf)

        base = sid * ROWS_PER_SID
        pltpu.sync_copy(zbuf, acc.at[pl.ds(base, CHUNK)])
        pltpu.sync_copy(zbuf, acc.at[pl.ds(base + CHUNK, CHUNK)])
        plsc.subcore_barrier()

        for p in range(3):
            pltpu.async_copy(
                node_hbm.at[pl.ds((wid + NW * p) * CHUNK, CHUNK)],
                rbuf.at[p], sem.at[p],
            )

        zv = jnp.zeros((16,), jnp.float32)
        gvec = jnp.full((16,), GARBAGE, jnp.int32)
        lane = lax.iota(jnp.int32, 16)

        def body(j2, carry):
            jA = 2 * j2
            j = 2 * j2 + 1
            bA = lax.rem(jA, 3)
            b = lax.rem(j, 3)

            @pl.when(jnp.logical_and(j2 >= 1, j2 <= 10))
            def _():
                cq = wid + NW * (jA + 2)
                bq = lax.rem(jA + 2, 3)
                pltpu.async_copy(
                    node_hbm.at[pl.ds(cq * CHUNK, CHUNK)], rbuf.at[bq], sem.at[bq]
                )

            pltpu.make_async_copy(
                node_hbm.at[pl.ds(0, CHUNK)], rbuf.at[bA], sem.at[bA]
            ).wait()
            scat = pltpu.async_copy(
                rbuf.at[bA], acc.at[ids_buf.at[jA]], sem_s, add=True
            )

            pltpu.make_async_copy(
                node_hbm.at[pl.ds(0, CHUNK)], rbuf.at[b], sem.at[b]
            ).wait()

            for v in range(NV + 2):
                idsC[pl.ds(v * 16, 16)] = gvec

            def group(g, c):
                avec = list(c[0:NV])
                kcur = c[NV]
                slot_v = slots_buf[j, pl.ds(g * 16, 16)]
                ids_v = ids_buf[j, pl.ds(g * 16, 16)]
                for r in range(16):
                    slot_r = slot_v[r]
                    same_r = slot_r == kcur
                    kcur = slot_r
                    idsC[pl.ds(slot_r, 16)] = jnp.where(lane == 0, ids_v[r], GARBAGE)
                    rows = [
                        rbuf[b, g * 16 + r, pl.ds(v * 16, 16)] for v in range(NV)
                    ]
                    avec = [
                        jnp.where(same_r, avec[v] + rows[v], rows[v])
                        for v in range(NV)
                    ]
                    for v in range(NV):
                        cbuf[slot_r, pl.ds(v * 16, 16)] = avec[v]
                return (*avec, kcur)

            init = (zv,) * NV + (jnp.int32(-1),)
            fin = lax.fori_loop(0, CHUNK // 16, group, init)
            kcnt = fin[NV] + 1

            @pl.when(kcnt <= 16)
            def _():
                ids16[pl.ds(0, 16)] = idsC[pl.ds(0, 16)]
                pltpu.sync_copy(cbuf.at[pl.ds(0, 16)], acc.at[ids16], add=True)

            @pl.when(jnp.logical_and(kcnt > 16, kcnt <= 32))
            def _():
                for v in range(2):
                    ids32[pl.ds(v * 16, 16)] = idsC[pl.ds(v * 16, 16)]
                pltpu.sync_copy(cbuf.at[pl.ds(0, 32)], acc.at[ids32], add=True)

            @pl.when(jnp.logical_and(kcnt > 32, kcnt <= 64))
            def _():
                for v in range(4):
                    ids64[pl.ds(v * 16, 16)] = idsC[pl.ds(v * 16, 16)]
                pltpu.sync_copy(cbuf.at[pl.ds(0, 64)], acc.at[ids64], add=True)

            @pl.when(kcnt > 64)
            def _():
                for v in range(NV):
                    ids128[pl.ds(v * 16, 16)] = idsC[pl.ds(v * 16, 16)]
                pltpu.sync_copy(cbuf, acc.at[ids128], add=True)

            scat.wait()

            @pl.when(j2 <= 10)
            def _():
                cq = wid + NW * (jA + 3)
                pltpu.async_copy(
                    node_hbm.at[pl.ds(cq * CHUNK, CHUNK)], rbuf.at[bA], sem.at[bA]
                )

            return carry

        lax.fori_loop(0, 12, body, 0)

        @pl.when(wid <= 12)
        def _():
            pltpu.sync_copy(
                node_hbm.at[pl.ds((wid + NW * (JMAX - 1)) * CHUNK, CHUNK)],
                rbuf.at[0],
            )
            pltpu.sync_copy(rbuf.at[0], acc.at[ids_buf.at[JMAX - 1]], add=True)

        @pl.when(wid == 13)
        def _():
            pltpu.sync_copy(
                node_hbm.at[pl.ds(NFULL * CHUNK, TAIL)], zbuf.at[pl.ds(0, TAIL)]
            )
            pltpu.sync_copy(zbuf, acc.at[ids_buf.at[JMAX - 1]], add=True)

        plsc.subcore_barrier()

        pltpu.sync_copy(
            acc.at[pl.ds(base, ROWS_PER_SID)],
            out_hbm.at[cid, pl.ds(base, ROWS_PER_SID)],
        )

    return seg_sum(node_feats, slots_t, idx_t, zrow)


def _tc_body(p_ref, g_ref, b_ref, w1_ref, b1_ref, w2_ref, b2_ref, o_ref):
    x = p_ref[0] + p_ref[1]
    mean = jnp.mean(x, axis=0, keepdims=True)
    xc = x - mean
    var = jnp.mean(xc * xc, axis=0, keepdims=True)
    gn = xc * lax.rsqrt(var + 1e-5) * g_ref[...] + b_ref[...]
    h = jnp.dot(gn, w1_ref[...], preferred_element_type=jnp.float32) + b1_ref[...]
    h = jnp.maximum(h, 0.0)
    o_ref[...] = jnp.dot(h, w2_ref[...], preferred_element_type=jnp.float32) + b2_ref[...]


def _tc_bn_mlp(partials, gamma, beta, W1, b1, W2, b2):
    return pl.pallas_call(
        _tc_body,
        out_shape=jax.ShapeDtypeStruct((G, H_OUT), jnp.float32),
    )(partials, gamma, beta, W1, b1, W2, b2)


def kernel(node_feats, segment_ids, gamma, beta, W1, b1, W2, b2):
    ids_i32 = segment_ids.astype(jnp.int32)
    ids_pad = jnp.zeros((NCH * CHUNK,), jnp.int32).at[:N].set(ids_i32)
    idc = ids_pad.reshape(NCH, CHUNK)
    bnd = jnp.concatenate(
        [jnp.ones((NCH, 1), jnp.int32),
         (idc[:, 1:] != idc[:, :-1]).astype(jnp.int32)],
        axis=1,
    )
    slots = jnp.cumsum(bnd, axis=1) - 1

    def arrange(t):
        return t.reshape(JMAX, NW, CHUNK).transpose(1, 0, 2)

    slots_t = arrange(slots)
    idx_t = arrange(idc)
    zrow = jnp.zeros((CHUNK, D), jnp.float32)

    partials = _sc_segment_sum(node_feats, slots_t, idx_t, zrow)

    return _tc_bn_mlp(
        partials,
        gamma.reshape(1, D),
        beta.reshape(1, D),
        W1,
        b1.reshape(1, D),
        W2,
        b2.reshape(1, H_OUT),
    )


# speedup vs baseline: 1.3995x; 1.3995x over previous
import functools

import jax
import jax.numpy as jnp
from jax import lax
from jax.experimental import pallas as pl
from jax.experimental.pallas import tpu as pltpu
from jax.experimental.pallas import tpu_sc as plsc

N = 100000
D = 128
G = 4096
H_OUT = 12

NC = 2
NS = 16
NW = NC * NS
CHUNK = 128
NFULL = N // CHUNK
TAIL = N - NFULL * CHUNK
JMAX = 25
ROWS_PER_SID = G // NS


def _sc_segment_sum(node_feats, idx_t, zrow):
    mesh = plsc.VectorSubcoreMesh(
        core_axis_name="c", subcore_axis_name="s", num_cores=NC, num_subcores=NS
    )

    @functools.partial(
        pl.kernel,
        out_type=jax.ShapeDtypeStruct((NC, G, D), jnp.float32),
        mesh=mesh,
        scratch_types=[
            pltpu.VMEM((JMAX, CHUNK), jnp.int32),
            pltpu.VMEM((4, CHUNK, D), jnp.float32),
            pltpu.VMEM((CHUNK, D), jnp.float32),
            pltpu.VMEM_SHARED((G, D), jnp.float32),
            pltpu.SemaphoreType.DMA((4,)),
            pltpu.SemaphoreType.DMA,
        ],
    )
    def seg_sum(node_hbm, idx_hbm, zrow_hbm, out_hbm, ids_buf, rbuf, zbuf, acc,
                sem, sem_s):
        cid = lax.axis_index("c")
        sid = lax.axis_index("s")
        wid = cid * NS + sid

        pltpu.sync_copy(idx_hbm.at[wid], ids_buf)
        pltpu.sync_copy(zrow_hbm, zbuf)

        base = sid * ROWS_PER_SID
        pltpu.sync_copy(zbuf, acc.at[pl.ds(base, CHUNK)])
        pltpu.sync_copy(zbuf, acc.at[pl.ds(base + CHUNK, CHUNK)])
        plsc.subcore_barrier()

        nfull = jnp.where(wid <= 12, JMAX, JMAX - 1)

        for p in range(2):
            pltpu.async_copy(
                node_hbm.at[pl.ds((wid + NW * p) * CHUNK, CHUNK)],
                rbuf.at[p], sem.at[p],
            )

        def body(j, carry):
            b = lax.rem(j, 4)

            @pl.when(j >= 2)
            def _():
                pltpu.make_async_copy(
                    rbuf.at[0], acc.at[ids_buf.at[0]], sem_s
                ).wait()

            @pl.when(j + 2 < nfull)
            def _():
                c2 = wid + NW * (j + 2)
                b2 = lax.rem(j + 2, 4)
                pltpu.async_copy(
                    node_hbm.at[pl.ds(c2 * CHUNK, CHUNK)], rbuf.at[b2], sem.at[b2]
                )

            pltpu.make_async_copy(
                node_hbm.at[pl.ds(0, CHUNK)], rbuf.at[b], sem.at[b]
            ).wait()
            pltpu.async_copy(rbuf.at[b], acc.at[ids_buf.at[j]], sem_s, add=True)
            return carry

        lax.fori_loop(0, nfull, body, 0)

        for _ in range(2):
            pltpu.make_async_copy(rbuf.at[0], acc.at[ids_buf.at[0]], sem_s).wait()

        @pl.when(wid == 13)
        def _():
            pltpu.sync_copy(
                node_hbm.at[pl.ds(NFULL * CHUNK, TAIL)], zbuf.at[pl.ds(0, TAIL)]
            )
            pltpu.sync_copy(zbuf, acc.at[ids_buf.at[JMAX - 1]], add=True)

        plsc.subcore_barrier()

        pltpu.sync_copy(
            acc.at[pl.ds(base, ROWS_PER_SID)],
            out_hbm.at[cid, pl.ds(base, ROWS_PER_SID)],
        )

    return seg_sum(node_feats, idx_t, zrow)


def _tc_body(p_ref, g_ref, b_ref, w1_ref, b1_ref, w2_ref, b2_ref, o_ref):
    x = p_ref[0] + p_ref[1]
    mean = jnp.mean(x, axis=0, keepdims=True)
    xc = x - mean
    var = jnp.mean(xc * xc, axis=0, keepdims=True)
    gn = xc * lax.rsqrt(var + 1e-5) * g_ref[...] + b_ref[...]
    h = jnp.dot(gn, w1_ref[...], preferred_element_type=jnp.float32) + b1_ref[...]
    h = jnp.maximum(h, 0.0)
    o_ref[...] = jnp.dot(h, w2_ref[...], preferred_element_type=jnp.float32) + b2_ref[...]


def _tc_bn_mlp(partials, gamma, beta, W1, b1, W2p, b2p):
    return pl.pallas_call(
        _tc_body,
        out_shape=jax.ShapeDtypeStruct((G, D), jnp.float32),
    )(partials, gamma, beta, W1, b1, W2p, b2p)


def kernel(node_feats, segment_ids, gamma, beta, W1, b1, W2, b2):
    ids32 = segment_ids.astype(jnp.int32)
    ids_pad = jnp.zeros((NW * JMAX * CHUNK,), jnp.int32).at[:N].set(ids32)
    idx_t = ids_pad.reshape(JMAX, NW, CHUNK).transpose(1, 0, 2)
    zrow = jnp.zeros((CHUNK, D), jnp.float32)

    partials = _sc_segment_sum(node_feats, idx_t, zrow)

    W2p = jnp.zeros((D, D), jnp.float32).at[:, :H_OUT].set(W2)
    b2p = jnp.zeros((1, D), jnp.float32).at[0, :H_OUT].set(b2)
    out = _tc_bn_mlp(
        partials,
        gamma.reshape(1, D),
        beta.reshape(1, D),
        W1,
        b1.reshape(1, D),
        W2p,
        b2p,
    )
    return out[:, :H_OUT]
